# baseline (speedup 1.0000x reference)
import jax
import jax.numpy as jnp
from jax.experimental import pallas as pl

_BLOCK = 3 * 1024 * 1024


def _copy_block(v_ref, o_ref):
    o_ref[...] = v_ref[...]


def kernel(values, indices):
    n = values.shape[0]
    grid = (pl.cdiv(n, _BLOCK),)
    return pl.pallas_call(
        _copy_block,
        grid=grid,
        in_specs=[pl.BlockSpec((_BLOCK,), lambda i: (i,))],
        out_specs=pl.BlockSpec((_BLOCK,), lambda i: (i,)),
        out_shape=jax.ShapeDtypeStruct(values.shape, values.dtype),
    )(values)


# speedup vs baseline: 1.0797x; 1.0797x over previous
import jax
import jax.numpy as jnp
from jax.experimental import pallas as pl

_BLOCK = 1536 * 1024


def _copy_block(v_ref, o_ref):
    o_ref[...] = v_ref[...]


def kernel(values, indices):
    n = values.shape[0]
    grid = (pl.cdiv(n, _BLOCK),)
    return pl.pallas_call(
        _copy_block,
        grid=grid,
        in_specs=[pl.BlockSpec((_BLOCK,), lambda i: (i,))],
        out_specs=pl.BlockSpec((_BLOCK,), lambda i: (i,)),
        out_shape=jax.ShapeDtypeStruct(values.shape, values.dtype),
    )(values)
